# baseline (speedup 1.0000x reference)
import functools

import jax
import jax.numpy as jnp
from jax import lax
from jax.experimental import pallas as pl
from jax.experimental.pallas import tpu as pltpu
from jax.experimental.pallas import tpu_sc as plsc

B = 16384
F = 26
D = 64
NC, NS = 2, 16
NW = NC * NS
E_PER_W = B // NW
CH = 32
NCHUNK = E_PER_W // CH
GI = 104
G = CH * F // GI
IDX_ROWS_PER_W = E_PER_W * F // GI
LANES = 16
KD = D // LANES


V = 100000
VB = 128
NBT = (V + VB - 1) // VB
V_PAD = NBT * VB
JMAX = (NBT + NW - 1) // NW
IDX_PIECE = 256


def _sc_prep(indices, table_t):
    mesh = plsc.VectorSubcoreMesh(core_axis_name="c", subcore_axis_name="s")
    RB = B // NW

    @functools.partial(
        pl.kernel,
        out_type=(jax.ShapeDtypeStruct((B * F,), jnp.int32),
                  jax.ShapeDtypeStruct((V_PAD * D,), jnp.float32)),
        mesh=mesh,
        scratch_types=[
            pltpu.VMEM((IDX_PIECE, F), jnp.int32),
            pltpu.VMEM((RB * F,), jnp.int32),
            pltpu.VMEM((2, D, VB), jnp.float32),
            pltpu.VMEM((2, VB * D), jnp.float32),
            pltpu.SemaphoreType.DMA,
            pltpu.SemaphoreType.DMA,
        ],
        compiler_params=pltpu.CompilerParams(use_tc_tiling_on_sc=True,
                                             needs_layout_passes=False),
    )
    def kp(idx_hbm, tt_hbm, idxout_hbm, tabout_hbm, idx_a, obuf, tin, tout,
           sem_in, sem_out):
        wid = lax.axis_index("s") * NC + lax.axis_index("c")

        r0 = wid * RB
        for piece in range(RB // IDX_PIECE):
            pltpu.sync_copy(idx_hbm.at[pl.ds(r0 + piece * IDX_PIECE,
                                             IDX_PIECE)], idx_a)

            def row_body(r, _):
                v0 = idx_a[r, pl.ds(0, LANES)]
                v1 = idx_a[r, pl.ds(F - LANES, LANES)]
                base = (piece * IDX_PIECE + r) * F
                obuf[pl.ds(base, LANES)] = v0
                obuf[pl.ds(base + F - LANES, LANES)] = v1
                return 0

            lax.fori_loop(0, IDX_PIECE, row_body, 0)
        pltpu.sync_copy(obuf, idxout_hbm.at[pl.ds(wid * RB * F, RB * F)])

        lane = jax.lax.iota(jnp.int32, LANES)
        didx = [lane + kk * LANES for kk in range(KD)]

        def tp_issue(blk, bslot):
            pltpu.async_copy(tt_hbm.at[:, pl.ds(blk * VB, VB)],
                             tin.at[bslot], sem_in)

        def tout_wait(blk, bslot):
            pltpu.make_async_copy(
                tout.at[bslot],
                tabout_hbm.at[pl.ds(blk * VB * D, VB * D)],
                sem_out).wait()

        def tp_block(blk, bslot):
            pltpu.make_async_copy(tt_hbm.at[:, pl.ds(blk * VB, VB)],
                                  tin.at[bslot], sem_in).wait()

            def col_body(vv, _):
                vidx = jnp.full((LANES,), vv, dtype=jnp.int32)
                for kk in range(KD):
                    col = plsc.load_gather(tin.at[bslot], [didx[kk], vidx])
                    tout[bslot, pl.ds(vv * D + kk * LANES, LANES)] = col
                return 0

            lax.fori_loop(0, VB, col_body, 0)
            pltpu.async_copy(tout.at[bslot],
                             tabout_hbm.at[pl.ds(blk * VB * D, VB * D)],
                             sem_out)

        tp_issue(wid, 0)
        tp_issue(wid + NW, 1)

        @pl.loop(0, JMAX - 1, step=2)
        def blk_loop(g):
            for bslot in range(2):
                j = g + bslot
                blk = wid + NW * j

                @pl.when(j >= 2)
                def _():
                    tout_wait(blk - 2 * NW, bslot)

                tp_block(blk, bslot)

                @pl.when(blk + 2 * NW < NBT)
                def _():
                    tp_issue(blk + 2 * NW, bslot)

        blk_last = wid + NW * (JMAX - 1)

        @pl.when(blk_last < NBT)
        def _():
            tout_wait(blk_last - 2 * NW, 0)
            tp_block(blk_last, 0)
            tout_wait(blk_last, 0)

        @pl.when(jnp.logical_not(blk_last < NBT))
        def _():
            tout_wait(blk_last - 2 * NW, 0)

        tout_wait(wid + NW * (JMAX - 2), 1)

    return kp(indices, table_t)


def _sc_gather_sum(idx1d, table):
    mesh = plsc.VectorSubcoreMesh(core_axis_name="c", subcore_axis_name="s")

    @functools.partial(
        pl.kernel,
        out_type=jax.ShapeDtypeStruct((B, D), jnp.float32),
        mesh=mesh,
        scratch_types=[
            pltpu.VMEM((2, CH * F), jnp.int32),
            pltpu.VMEM((2, CH * F, D), jnp.float32),
            pltpu.VMEM((2, CH, D), jnp.float32),
            pltpu.SemaphoreType.DMA,
            pltpu.SemaphoreType.DMA,
        ],
        compiler_params=pltpu.CompilerParams(use_tc_tiling_on_sc=False),
    )
    def k(idx_hbm, table_hbm, out_hbm, idx_v, rows_v, out_v, sem0, sem1):
        wid = lax.axis_index("s") * NC + lax.axis_index("c")
        out_base = wid * E_PER_W
        sems = (sem0, sem1)

        def issue(c, bslot):
            pltpu.sync_copy(
                idx_hbm.at[pl.ds((out_base + c * CH) * F, CH * F)],
                idx_v.at[bslot])
            for j in range(G):
                pltpu.async_copy(
                    table_hbm.at[idx_v.at[bslot, pl.ds(j * GI, GI)]],
                    rows_v.at[bslot, pl.ds(j * GI, GI)],
                    sems[bslot],
                )

        def drain(bslot):
            for j in range(G):
                pltpu.make_async_copy(
                    table_hbm.at[idx_v.at[bslot, pl.ds(j * GI, GI)]],
                    rows_v.at[bslot, pl.ds(j * GI, GI)],
                    sems[bslot],
                ).wait()

        def accumulate(c, bslot):
            def ent_body(e, _):
                r0 = e * F
                for kk in range(KD):
                    acc = rows_v[bslot, r0, pl.ds(kk * LANES, LANES)]
                    for f in range(1, F):
                        acc = acc + rows_v[bslot, r0 + f, pl.ds(kk * LANES, LANES)]
                    out_v[bslot, e, pl.ds(kk * LANES, LANES)] = acc
                return 0

            lax.fori_loop(0, CH, ent_body, 0)
            pltpu.sync_copy(out_v.at[bslot],
                            out_hbm.at[pl.ds(out_base + c * CH, CH)])

        issue(0, 0)
        issue(1, 1)

        @pl.loop(0, NCHUNK, step=2)
        def chunk_body(g):
            for bslot in range(2):
                c = g + bslot
                drain(bslot)
                accumulate(c, bslot)

                @pl.when(c + 2 < NCHUNK)
                def _():
                    issue(c + 2, bslot)

    return k(idx1d, table)


def _tc_mlp(summed, W, b):
    BM = 2048

    def body(x_ref, w_ref, b_ref, o_ref):
        y = jnp.dot(x_ref[...], w_ref[...], preferred_element_type=jnp.float32)
        o_ref[...] = jnp.maximum(y + b_ref[...], 0.0)

    return pl.pallas_call(
        body,
        grid=(B // BM,),
        in_specs=[
            pl.BlockSpec((BM, D), lambda i: (i, 0)),
            pl.BlockSpec((D, D), lambda i: (0, 0)),
            pl.BlockSpec((1, D), lambda i: (0, 0)),
        ],
        out_specs=pl.BlockSpec((BM, D), lambda i: (i, 0)),
        out_shape=jax.ShapeDtypeStruct((B, D), jnp.float32),
    )(summed, W, b.reshape(1, D))


def kernel(indices, table, W, b):
    idx1d, tablin = _sc_prep(indices, table.T)
    summed = _sc_gather_sum(idx1d, tablin.reshape(V_PAD, D))
    return _tc_mlp(summed, W, b)


# speedup vs baseline: 1.5957x; 1.5957x over previous
import functools

import jax
import jax.numpy as jnp
from jax import lax
from jax.experimental import pallas as pl
from jax.experimental.pallas import tpu as pltpu
from jax.experimental.pallas import tpu_sc as plsc

B = 16384
F = 26
D = 64
NC, NS = 2, 16
NW = NC * NS
E_PER_W = B // NW
CH = 32
NCHUNK = E_PER_W // CH
GI = 104
G = CH * F // GI
LANES = 16
KD = D // LANES


def _sc_flatten_idx(indices):
    mesh = plsc.VectorSubcoreMesh(core_axis_name="c", subcore_axis_name="s")
    RB = B // NW

    @functools.partial(
        pl.kernel,
        out_type=jax.ShapeDtypeStruct((B * F,), jnp.int32),
        mesh=mesh,
        scratch_types=[
            pltpu.VMEM((RB, F), jnp.int32),
            pltpu.VMEM((RB * F,), jnp.int32),
        ],
        compiler_params=pltpu.CompilerParams(use_tc_tiling_on_sc=True),
    )
    def ka(idx_hbm, out_hbm, idx_a, obuf):
        wid = lax.axis_index("s") * NC + lax.axis_index("c")
        r0 = wid * RB
        pltpu.sync_copy(idx_hbm.at[pl.ds(r0, RB)], idx_a)

        def row_body(r, _):
            v0 = idx_a[r, pl.ds(0, LANES)]
            v1 = idx_a[r, pl.ds(F - LANES, LANES)]
            base = r * F
            obuf[pl.ds(base, LANES)] = v0
            obuf[pl.ds(base + F - LANES, LANES)] = v1
            return 0

        lax.fori_loop(0, RB, row_body, 0)
        pltpu.sync_copy(obuf, out_hbm.at[pl.ds(wid * RB * F, RB * F)])

    return ka(indices)


def _sc_gather_sum(idx1d, table):
    mesh = plsc.VectorSubcoreMesh(core_axis_name="c", subcore_axis_name="s")

    @functools.partial(
        pl.kernel,
        out_type=jax.ShapeDtypeStruct((B, D), jnp.float32),
        mesh=mesh,
        scratch_types=[
            pltpu.VMEM((2, CH * F), jnp.int32),
            pltpu.VMEM((2, CH * F, D), jnp.float32),
            pltpu.VMEM((2, CH, D), jnp.float32),
            pltpu.SemaphoreType.DMA,
            pltpu.SemaphoreType.DMA,
        ],
        compiler_params=pltpu.CompilerParams(use_tc_tiling_on_sc=False),
    )
    def k(idx_hbm, table_hbm, out_hbm, idx_v, rows_v, out_v, sem0, sem1):
        wid = lax.axis_index("s") * NC + lax.axis_index("c")
        out_base = wid * E_PER_W
        sems = (sem0, sem1)

        def issue(c, bslot):
            pltpu.sync_copy(
                idx_hbm.at[pl.ds((out_base + c * CH) * F, CH * F)],
                idx_v.at[bslot])
            pltpu.async_copy(
                table_hbm.at[idx_v.at[bslot]],
                rows_v.at[bslot],
                sems[bslot],
            )

        def drain(bslot):
            pltpu.make_async_copy(
                table_hbm.at[idx_v.at[bslot]],
                rows_v.at[bslot],
                sems[bslot],
            ).wait()

        def accumulate(c, bslot):
            def ent_body(e, _):
                r0 = e * F
                for kk in range(KD):
                    acc = rows_v[bslot, r0, pl.ds(kk * LANES, LANES)]
                    for f in range(1, F):
                        acc = acc + rows_v[bslot, r0 + f, pl.ds(kk * LANES, LANES)]
                    out_v[bslot, e, pl.ds(kk * LANES, LANES)] = acc
                return 0

            lax.fori_loop(0, CH, ent_body, 0)
            pltpu.sync_copy(out_v.at[bslot],
                            out_hbm.at[pl.ds(out_base + c * CH, CH)])

        issue(0, 0)
        issue(1, 1)

        @pl.loop(0, NCHUNK, step=2)
        def chunk_body(g):
            for bslot in range(2):
                c = g + bslot
                drain(bslot)
                accumulate(c, bslot)

                @pl.when(c + 2 < NCHUNK)
                def _():
                    issue(c + 2, bslot)

    return k(idx1d, table)


def _tc_mlp(summed, W, b):
    BM = 2048

    def body(x_ref, w_ref, b_ref, o_ref):
        y = jnp.dot(x_ref[...], w_ref[...], preferred_element_type=jnp.float32)
        o_ref[...] = jnp.maximum(y + b_ref[...], 0.0)

    return pl.pallas_call(
        body,
        grid=(B // BM,),
        in_specs=[
            pl.BlockSpec((BM, D), lambda i: (i, 0)),
            pl.BlockSpec((D, D), lambda i: (0, 0)),
            pl.BlockSpec((1, D), lambda i: (0, 0)),
        ],
        out_specs=pl.BlockSpec((BM, D), lambda i: (i, 0)),
        out_shape=jax.ShapeDtypeStruct((B, D), jnp.float32),
    )(summed, W, b.reshape(1, D))


def kernel(indices, table, W, b):
    idx1d = _sc_flatten_idx(indices)
    summed = _sc_gather_sum(idx1d, table)
    return _tc_mlp(summed, W, b)


# speedup vs baseline: 1.7762x; 1.1131x over previous
import functools

import jax
import jax.numpy as jnp
from jax import lax
from jax.experimental import pallas as pl
from jax.experimental.pallas import tpu as pltpu
from jax.experimental.pallas import tpu_sc as plsc

B = 16384
F = 26
D = 64
NC, NS = 2, 16
NW = NC * NS
E_PER_W = B // NW
CH = 32
NCHUNK = E_PER_W // CH
GI = 104
G = CH * F // GI
LANES = 16
KD = D // LANES


def _sc_flatten_idx(indices):
    mesh = plsc.VectorSubcoreMesh(core_axis_name="c", subcore_axis_name="s")
    RB = B // NW

    @functools.partial(
        pl.kernel,
        out_type=jax.ShapeDtypeStruct((B * F,), jnp.int32),
        mesh=mesh,
        scratch_types=[
            pltpu.VMEM((RB, F), jnp.int32),
            pltpu.VMEM((RB * F,), jnp.int32),
        ],
        compiler_params=pltpu.CompilerParams(use_tc_tiling_on_sc=True),
    )
    def ka(idx_hbm, out_hbm, idx_a, obuf):
        wid = lax.axis_index("s") * NC + lax.axis_index("c")
        r0 = wid * RB
        pltpu.sync_copy(idx_hbm.at[pl.ds(r0, RB)], idx_a)

        def row_body(r, _):
            v0 = idx_a[r, pl.ds(0, LANES)]
            v1 = idx_a[r, pl.ds(F - LANES, LANES)]
            base = r * F
            obuf[pl.ds(base, LANES)] = v0
            obuf[pl.ds(base + F - LANES, LANES)] = v1
            return 0

        lax.fori_loop(0, RB, row_body, 0)
        pltpu.sync_copy(obuf, out_hbm.at[pl.ds(wid * RB * F, RB * F)])

    return ka(indices)


def _sc_gather_sum(idx1d, table):
    mesh = plsc.VectorSubcoreMesh(core_axis_name="c", subcore_axis_name="s")

    @functools.partial(
        pl.kernel,
        out_type=jax.ShapeDtypeStruct((B, D), jnp.float32),
        mesh=mesh,
        scratch_types=[
            pltpu.VMEM((2, CH * F), jnp.int32),
            pltpu.VMEM((2, CH * F, D), jnp.float32),
            pltpu.VMEM((2, CH, D), jnp.float32),
            pltpu.SemaphoreType.DMA,
            pltpu.SemaphoreType.DMA,
        ],
        compiler_params=pltpu.CompilerParams(use_tc_tiling_on_sc=False),
    )
    def k(idx_hbm, table_hbm, out_hbm, idx_v, rows_v, out_v, sem0, sem1):
        wid = lax.axis_index("s") * NC + lax.axis_index("c")
        out_base = wid * E_PER_W
        sems = (sem0, sem1)

        def issue(c, bslot):
            pltpu.sync_copy(
                idx_hbm.at[pl.ds((out_base + c * CH) * F, CH * F)],
                idx_v.at[bslot])
            pltpu.async_copy(
                table_hbm.at[idx_v.at[bslot]],
                rows_v.at[bslot],
                sems[bslot],
            )

        def drain(bslot):
            pltpu.make_async_copy(
                table_hbm.at[idx_v.at[bslot]],
                rows_v.at[bslot],
                sems[bslot],
            ).wait()

        def accumulate(c, bslot):
            def ent_body(e, _):
                r0 = e * F
                for kk in range(KD):
                    vals = [rows_v[bslot, r0 + f, pl.ds(kk * LANES, LANES)]
                            for f in range(F)]
                    while len(vals) > 1:
                        nxt = [vals[i] + vals[i + 1]
                               for i in range(0, len(vals) - 1, 2)]
                        if len(vals) % 2:
                            nxt.append(vals[-1])
                        vals = nxt
                    out_v[bslot, e, pl.ds(kk * LANES, LANES)] = vals[0]
                return 0

            lax.fori_loop(0, CH, ent_body, 0)
            pltpu.sync_copy(out_v.at[bslot],
                            out_hbm.at[pl.ds(out_base + c * CH, CH)])

        issue(0, 0)
        issue(1, 1)

        @pl.loop(0, NCHUNK, step=2)
        def chunk_body(g):
            for bslot in range(2):
                c = g + bslot
                drain(bslot)
                accumulate(c, bslot)

                @pl.when(c + 2 < NCHUNK)
                def _():
                    issue(c + 2, bslot)

    return k(idx1d, table)


def _tc_mlp(summed, W, b):
    BM = 2048

    def body(x_ref, w_ref, b_ref, o_ref):
        y = jnp.dot(x_ref[...], w_ref[...], preferred_element_type=jnp.float32)
        o_ref[...] = jnp.maximum(y + b_ref[...], 0.0)

    return pl.pallas_call(
        body,
        grid=(B // BM,),
        in_specs=[
            pl.BlockSpec((BM, D), lambda i: (i, 0)),
            pl.BlockSpec((D, D), lambda i: (0, 0)),
            pl.BlockSpec((1, D), lambda i: (0, 0)),
        ],
        out_specs=pl.BlockSpec((BM, D), lambda i: (i, 0)),
        out_shape=jax.ShapeDtypeStruct((B, D), jnp.float32),
    )(summed, W, b.reshape(1, D))


def kernel(indices, table, W, b):
    idx1d = _sc_flatten_idx(indices)
    summed = _sc_gather_sum(idx1d, table)
    return _tc_mlp(summed, W, b)
